# Initial kernel scaffold; baseline (speedup 1.0000x reference)
import jax
import jax.numpy as jnp
from jax.experimental import pallas as pl


def kernel(states, actions, core0, core1, core2, core3, core4, core5, core6):
    raise NotImplementedError("write your pallas kernel here")


# speedup vs baseline: 13.0583x; 13.0583x over previous
import functools

import jax
import jax.numpy as jnp
from jax import lax
from jax.experimental import pallas as pl
from jax.experimental.pallas import tpu as pltpu
from jax.experimental.pallas import tpu_sc as plsc

B = 16384
R = 8
V = 64
NDIMS = 7
NC, NS, L = 2, 16, 16
NW = NC * NS
BPW = B // NW
GROUPS = BPW // L


def _tt_body(idx_hbm, t0_hbm, t1_hbm, t2_hbm, t3_hbm, t4_hbm, t5_hbm,
             t6_hbm, out_hbm, idx_v, t0_v, t1_v, t2_v, t3_v, t4_v, t5_v,
             t6_v, out_v):
    wid = lax.axis_index("s") * NC + lax.axis_index("c")
    base = wid * BPW

    pltpu.sync_copy(t0_hbm, t0_v)
    pltpu.sync_copy(t1_hbm, t1_v)
    pltpu.sync_copy(t2_hbm, t2_v)
    pltpu.sync_copy(t3_hbm, t3_v)
    pltpu.sync_copy(t4_hbm, t4_v)
    pltpu.sync_copy(t5_hbm, t5_v)
    pltpu.sync_copy(t6_hbm, t6_v)
    for k in range(NDIMS):
        pltpu.sync_copy(idx_hbm.at[pl.ds(k * B + base, BPW)],
                        idx_v.at[pl.ds(k * BPW, BPW)])

    tmid = [t1_v, t2_v, t3_v, t4_v, t5_v]

    def group(g, carry):
        o = g * L
        i0 = idx_v[pl.ds(0 * BPW + o, L)] * R
        res = [plsc.load_gather(t0_v, [i0 + j]) for j in range(R)]
        for k in range(1, 6):
            ik = idx_v[pl.ds(k * BPW + o, L)] * (R * R)
            tk = tmid[k - 1]
            new = []
            for l in range(R):
                acc = res[0] * plsc.load_gather(tk, [ik + l])
                for j in range(1, R):
                    acc = acc + res[j] * plsc.load_gather(tk, [ik + (j * R + l)])
                new.append(acc)
            res = new
        i6 = idx_v[pl.ds(6 * BPW + o, L)] * R
        q = res[0] * plsc.load_gather(t6_v, [i6 + 0])
        for j in range(1, R):
            q = q + res[j] * plsc.load_gather(t6_v, [i6 + j])
        out_v[pl.ds(o, L)] = q
        return carry

    lax.fori_loop(0, GROUPS, group, 0, unroll=False)
    pltpu.sync_copy(out_v, out_hbm.at[pl.ds(base, BPW)])


_tt_gather = functools.partial(
    pl.kernel,
    out_type=jax.ShapeDtypeStruct((B,), jnp.float32),
    mesh=plsc.VectorSubcoreMesh(core_axis_name="c", subcore_axis_name="s",
                                num_cores=NC, num_subcores=NS),
    compiler_params=pltpu.CompilerParams(needs_layout_passes=False),
    scratch_types=[
        pltpu.VMEM((NDIMS * BPW,), jnp.int32),
        pltpu.VMEM((V * R,), jnp.float32),
        pltpu.VMEM((V * R * R,), jnp.float32),
        pltpu.VMEM((V * R * R,), jnp.float32),
        pltpu.VMEM((V * R * R,), jnp.float32),
        pltpu.VMEM((V * R * R,), jnp.float32),
        pltpu.VMEM((V * R * R,), jnp.float32),
        pltpu.VMEM((V * R,), jnp.float32),
        pltpu.VMEM((BPW,), jnp.float32),
    ],
)(_tt_body)


def kernel(states, actions, core0, core1, core2, core3, core4, core5, core6):
    idx_flat = jnp.concatenate([states.T, actions.T], axis=0).reshape(-1)
    t0 = core0.reshape(-1)
    tmid = [jnp.transpose(c, (1, 0, 2)).reshape(-1)
            for c in (core1, core2, core3, core4, core5)]
    t6 = jnp.transpose(core6, (1, 0, 2)).reshape(-1)
    return _tt_gather(idx_flat, t0, *tmid, t6)


# speedup vs baseline: 35.4485x; 2.7146x over previous
import functools

import jax
import jax.numpy as jnp
from jax import lax
from jax.experimental import pallas as pl
from jax.experimental.pallas import tpu as pltpu
from jax.experimental.pallas import tpu_sc as plsc

B = 16384
R = 8
V = 64
NDIMS = 7
NC, NS, L = 2, 16, 16
NW = NC * NS
BPW = B // NW
GROUPS = BPW // L
SE = R + 1
SM = R * R + 1


def _tt_body(idx_hbm, t0_hbm, t1_hbm, t2_hbm, t3_hbm, t4_hbm, t5_hbm,
             t6_hbm, out_hbm, idx_v, t0_v, t1_v, t2_v, t3_v, t4_v, t5_v,
             t6_v, out_v, sem):
    wid = lax.axis_index("s") * NC + lax.axis_index("c")
    base = wid * BPW

    copies = [
        pltpu.async_copy(t0_hbm, t0_v, sem),
        pltpu.async_copy(t1_hbm, t1_v, sem),
        pltpu.async_copy(t2_hbm, t2_v, sem),
        pltpu.async_copy(t3_hbm, t3_v, sem),
        pltpu.async_copy(t4_hbm, t4_v, sem),
        pltpu.async_copy(t5_hbm, t5_v, sem),
        pltpu.async_copy(t6_hbm, t6_v, sem),
        pltpu.async_copy(idx_hbm.at[pl.ds(wid * (NDIMS * BPW), NDIMS * BPW)],
                         idx_v, sem),
    ]
    for c in copies:
        c.wait()

    tmid = [t1_v, t2_v, t3_v, t4_v, t5_v]

    def group(g, carry):
        o = g * L
        i0 = idx_v[pl.ds(0 * BPW + o, L)] * SE
        res = [plsc.load_gather(t0_v, [i0 + j]) for j in range(R)]
        for k in range(1, 6):
            ik = idx_v[pl.ds(k * BPW + o, L)] * SM
            tk = tmid[k - 1]
            new = []
            for l in range(R):
                acc = res[0] * plsc.load_gather(tk, [ik + l])
                for j in range(1, R):
                    acc = acc + res[j] * plsc.load_gather(tk, [ik + (j * R + l)])
                new.append(acc)
            res = new
        i6 = idx_v[pl.ds(6 * BPW + o, L)] * SE
        q = res[0] * plsc.load_gather(t6_v, [i6 + 0])
        for j in range(1, R):
            q = q + res[j] * plsc.load_gather(t6_v, [i6 + j])
        out_v[pl.ds(o, L)] = q
        return carry

    lax.fori_loop(0, GROUPS, group, 0, unroll=False)
    pltpu.sync_copy(out_v, out_hbm.at[pl.ds(base, BPW)])


_tt_gather = functools.partial(
    pl.kernel,
    out_type=jax.ShapeDtypeStruct((B,), jnp.float32),
    mesh=plsc.VectorSubcoreMesh(core_axis_name="c", subcore_axis_name="s",
                                num_cores=NC, num_subcores=NS),
    compiler_params=pltpu.CompilerParams(needs_layout_passes=False),
    scratch_types=[
        pltpu.VMEM((NDIMS * BPW,), jnp.int32),
        pltpu.VMEM((V * SE,), jnp.float32),
        pltpu.VMEM((V * SM,), jnp.float32),
        pltpu.VMEM((V * SM,), jnp.float32),
        pltpu.VMEM((V * SM,), jnp.float32),
        pltpu.VMEM((V * SM,), jnp.float32),
        pltpu.VMEM((V * SM,), jnp.float32),
        pltpu.VMEM((V * SE,), jnp.float32),
        pltpu.VMEM((BPW,), jnp.float32),
        pltpu.SemaphoreType.DMA,
    ],
)(_tt_body)


def _pad_rows(t, stride):
    return jnp.pad(t, ((0, 0), (0, stride - t.shape[1]))).reshape(-1)


def kernel(states, actions, core0, core1, core2, core3, core4, core5, core6):
    idxp = (jnp.concatenate([states.T, actions.T], axis=0)
            .reshape(NDIMS, NW, BPW).transpose(1, 0, 2).reshape(-1))
    t0 = _pad_rows(core0.reshape(V, R), SE)
    tmid = [_pad_rows(jnp.transpose(c, (1, 0, 2)).reshape(V, R * R), SM)
            for c in (core1, core2, core3, core4, core5)]
    t6 = _pad_rows(jnp.transpose(core6, (1, 0, 2)).reshape(V, R), SE)
    return _tt_gather(idxp, t0, *tmid, t6)
